# baseline (speedup 1.0000x reference)
import numpy as np
import jax
import jax.numpy as jnp
from jax.experimental import pallas as pl

_B, _C, _T = 128, 128, 2048
_NL = 527


def _mix_matrix():
    rs = np.random.RandomState(seed=1234)
    rs.uniform()
    rs.uniform()
    lam = rs.beta(0.3, 0.3)
    perm = rs.permutation(_B)
    m = np.zeros((_B, _B), np.float32)
    m[np.arange(_B), np.arange(_B)] += np.float32(1.0 - lam)
    m[np.arange(_B), perm] += np.float32(lam)
    return m


_MIX = _mix_matrix()


_TBLK = 2048
_NBLK = (_C * _T) // _TBLK


def _body(m_ref, x_ref, y_ref, ox_ref, oy_ref):
    m = m_ref[...]
    ox_ref[...] = jnp.dot(m, x_ref[...], preferred_element_type=jnp.float32)

    @pl.when(pl.program_id(0) == 0)
    def _():
        oy_ref[...] = jnp.dot(m, y_ref[...], preferred_element_type=jnp.float32)


def kernel(x, y):
    x2 = x.reshape(_B, _C * _T)
    ox, oy = pl.pallas_call(
        _body,
        grid=(_NBLK,),
        in_specs=[
            pl.BlockSpec((_B, _B), lambda c: (0, 0)),
            pl.BlockSpec((_B, _TBLK), lambda c: (0, c)),
            pl.BlockSpec((_B, _NL), lambda c: (0, 0)),
        ],
        out_specs=[
            pl.BlockSpec((_B, _TBLK), lambda c: (0, c)),
            pl.BlockSpec((_B, _NL), lambda c: (0, 0)),
        ],
        out_shape=[
            jax.ShapeDtypeStruct((_B, _C * _T), jnp.float32),
            jax.ShapeDtypeStruct((_B, _NL), jnp.float32),
        ],
    )(jnp.asarray(_MIX), x2, y)
    return (ox.reshape(_B, _C, _T), oy)


# speedup vs baseline: 2.2533x; 2.2533x over previous
import numpy as np
import jax
import jax.numpy as jnp
from jax.experimental import pallas as pl
from jax.experimental.pallas import tpu as pltpu

_B, _C, _T = 128, 128, 2048
_NL = 527


def _mix_plan():
    rs = np.random.RandomState(seed=1234)
    rs.uniform()
    rs.uniform()
    lam = rs.beta(0.3, 0.3)
    perm = rs.permutation(_B)
    src, dst, flag = [], [], []
    visited = np.zeros(_B, bool)
    for s in range(_B):
        if visited[s]:
            continue
        src.append(s)
        dst.append(s)
        flag.append(0)
        e = s
        while True:
            visited[e] = True
            src.append(int(perm[e]))
            dst.append(int(e))
            flag.append(1)
            if perm[e] == s:
                break
            e = int(perm[e])
    return (
        float(lam),
        np.asarray(src, np.int32),
        np.asarray(dst, np.int32),
        np.asarray(flag, np.int32),
    )


_LAM, _SRC, _DST, _FLAG = _mix_plan()
_G = len(_SRC)


def _body(src_ref, dst_ref, flag_ref, x_ref, y_ref, ox_ref, oy_ref, xs_ref, ys_ref):
    g = pl.program_id(0)
    xv = x_ref[...]
    yv = y_ref[...]

    @pl.when(flag_ref[g] == 1)
    def _():
        ox_ref[...] = (1.0 - _LAM) * xs_ref[...] + _LAM * xv
        oy_ref[...] = (1.0 - _LAM) * ys_ref[...] + _LAM * yv

    xs_ref[...] = xv
    ys_ref[...] = yv


def kernel(x, y):
    y3 = y.reshape(_B, 1, _NL)
    grid_spec = pltpu.PrefetchScalarGridSpec(
        num_scalar_prefetch=3,
        grid=(_G,),
        in_specs=[
            pl.BlockSpec((1, _C, _T), lambda g, src, dst, flag: (src[g], 0, 0)),
            pl.BlockSpec((1, 1, _NL), lambda g, src, dst, flag: (src[g], 0, 0)),
        ],
        out_specs=[
            pl.BlockSpec((1, _C, _T), lambda g, src, dst, flag: (dst[g], 0, 0)),
            pl.BlockSpec((1, 1, _NL), lambda g, src, dst, flag: (dst[g], 0, 0)),
        ],
        scratch_shapes=[
            pltpu.VMEM((1, _C, _T), jnp.float32),
            pltpu.VMEM((1, 1, _NL), jnp.float32),
        ],
    )
    ox, oy = pl.pallas_call(
        _body,
        grid_spec=grid_spec,
        out_shape=[
            jax.ShapeDtypeStruct((_B, _C, _T), jnp.float32),
            jax.ShapeDtypeStruct((_B, 1, _NL), jnp.float32),
        ],
    )(jnp.asarray(_SRC), jnp.asarray(_DST), jnp.asarray(_FLAG), x, y3)
    return (ox, oy.reshape(_B, _NL))
